# Initial kernel scaffold; baseline (speedup 1.0000x reference)
import jax
import jax.numpy as jnp
from jax.experimental import pallas as pl


def kernel(event_embed, labels, bW1, bb1, bW2, bb2, bW3, bb3, cW1, cb1, cW2, cb2, cW3, cb3, W1, root1, bias1, W2, root2, bias2):
    raise NotImplementedError("write your pallas kernel here")


# speedup vs baseline: 135.5121x; 135.5121x over previous
import functools

import jax
import jax.numpy as jnp
from jax.experimental import pallas as pl
from jax.experimental.pallas import tpu as pltpu

N = 256
R = 7
D = 768
H = 150
NREL = 6


def _conv_kernel(x_ref, labpad_ref, root_ref, bias_ref, w_ref, out_ref, *, relu):
    r = pl.program_id(0)
    labpad = labpad_ref[...]
    shifted = jnp.concatenate(
        [jnp.full((N, 1), 6, jnp.int32), labpad[:, : N - 1]], axis=1
    )
    ii = jax.lax.broadcasted_iota(jnp.int32, (N, N), 0)
    kk = jax.lax.broadcasted_iota(jnp.int32, (N, N), 1)
    lab_full = jnp.where(kk < ii, labpad, jnp.where(kk > ii, shifted, 6))
    m = (lab_full == r).astype(jnp.float32)
    x = x_ref[...]
    sums = jax.lax.dot_general(
        m, x, (((0,), (0,)), ((), ())), preferred_element_type=jnp.float32
    )
    cnt = jnp.sum(m, axis=0)
    mean = sums / jnp.maximum(cnt, 1.0)[:, None]
    contrib = jnp.dot(mean, w_ref[0], preferred_element_type=jnp.float32)

    @pl.when(r == 0)
    def _init():
        base = jnp.dot(x, root_ref[...], preferred_element_type=jnp.float32)
        out_ref[...] = base + bias_ref[...] + contrib

    @pl.when(r > 0)
    def _acc():
        out_ref[...] = out_ref[...] + contrib

    if relu:
        @pl.when(r == NREL - 1)
        def _act():
            out_ref[...] = jnp.maximum(out_ref[...], 0.0)


def _rgcn_conv(x, labpad, w, root, bias, relu):
    return pl.pallas_call(
        functools.partial(_conv_kernel, relu=relu),
        grid=(NREL,),
        in_specs=[
            pl.BlockSpec((N, D), lambda r: (0, 0)),
            pl.BlockSpec((N, N), lambda r: (0, 0)),
            pl.BlockSpec((D, D), lambda r: (0, 0)),
            pl.BlockSpec((1, D), lambda r: (0, 0)),
            pl.BlockSpec((1, D, D), lambda r: (r, 0, 0)),
        ],
        out_specs=pl.BlockSpec((N, D), lambda r: (0, 0)),
        out_shape=jax.ShapeDtypeStruct((N, D), jnp.float32),
    )(x, labpad, root, bias, w)


BI = 32


def _pair_kernel(x_ref, w1a_ref, w1b_ref, b1_ref, w2_ref, b2_ref, w3_ref,
                 b3_ref, out_ref, v_ref):
    i = pl.program_id(0)

    @pl.when(i == 0)
    def _precompute_v():
        v_ref[...] = jnp.dot(
            x_ref[...], w1b_ref[...], preferred_element_type=jnp.float32
        )

    xb = x_ref[pl.ds(i * BI, BI), :]
    u = jnp.dot(xb, w1a_ref[...], preferred_element_type=jnp.float32)
    v = v_ref[...]
    h1 = jnp.maximum(u[:, None, :] + v[None, :, :] + b1_ref[...][None, :, :], 0.0)
    h1 = h1.reshape(BI * N, H)
    h2 = jnp.maximum(
        jnp.dot(h1, w2_ref[...], preferred_element_type=jnp.float32) + b2_ref[...],
        0.0,
    )
    s = jnp.dot(h2, w3_ref[...], preferred_element_type=jnp.float32) + b3_ref[...]
    s = s.reshape(BI, N, R)
    ig = i * BI + jax.lax.broadcasted_iota(jnp.int32, (BI, N - 1, 1), 0)
    jj = jax.lax.broadcasted_iota(jnp.int32, (BI, N - 1, 1), 1)
    out_ref[...] = jnp.where(jj < ig, s[:, : N - 1, :], s[:, 1:, :])


def _pair_mlp(x, w1a, w1b, b1, w2, b2, w3, b3):
    return pl.pallas_call(
        _pair_kernel,
        grid=(N // BI,),
        in_specs=[
            pl.BlockSpec((N, D), lambda i: (0, 0)),
            pl.BlockSpec((D, H), lambda i: (0, 0)),
            pl.BlockSpec((D, H), lambda i: (0, 0)),
            pl.BlockSpec((1, H), lambda i: (0, 0)),
            pl.BlockSpec((H, H), lambda i: (0, 0)),
            pl.BlockSpec((1, H), lambda i: (0, 0)),
            pl.BlockSpec((H, R), lambda i: (0, 0)),
            pl.BlockSpec((1, R), lambda i: (0, 0)),
        ],
        out_specs=pl.BlockSpec((BI, N - 1, R), lambda i: (i, 0, 0)),
        out_shape=jax.ShapeDtypeStruct((N, N - 1, R), jnp.float32),
        scratch_shapes=[pltpu.VMEM((N, H), jnp.float32)],
    )(x, w1a, w1b, b1, w2, b2, w3, b3)


def kernel(event_embed, labels, bW1, bb1, bW2, bb2, bW3, bb3,
           cW1, cb1, cW2, cb2, cW3, cb3,
           W1, root1, bias1, W2, root2, bias2):
    x = event_embed[0]
    labpad = jnp.concatenate(
        [labels.reshape(N, N - 1), jnp.full((N, 1), 6, jnp.int32)], axis=1
    )
    h = _rgcn_conv(x, labpad, W1, root1, bias1.reshape(1, D), relu=True)
    out = _rgcn_conv(h, labpad, W2, root2, bias2.reshape(1, D), relu=False)
    scores = _pair_mlp(
        out,
        cW1[:D], cW1[D:], cb1.reshape(1, H),
        cW2, cb2.reshape(1, H),
        cW3, cb3.reshape(1, R),
    )
    return scores.reshape(1, N * (N - 1), R)
